# baseline (speedup 1.0000x reference)
import functools

import jax
import jax.numpy as jnp
from jax.experimental import pallas as pl
from jax.experimental.pallas import tpu as pltpu

S = 2048
E = 64
D_MODEL = 768
D_FF = 3072
T = 512
NB = S // T
NT = NB + E - 1


def _route_kernel(x_ref, rw_ref, gx_ref, dest_ref, counts_ref):
    x = x_ref[...]
    rw = rw_ref[...]
    logits = jax.lax.dot_general(
        x, rw, (((1,), (1,)), ((), ())), preferred_element_type=jnp.float32)
    m = jnp.max(logits, axis=1, keepdims=True)
    e_iota = jax.lax.broadcasted_iota(jnp.int32, (S, E), 1)
    expert = jnp.min(jnp.where(logits == m, e_iota, E), axis=1, keepdims=True)

    onehot = (e_iota == expert).astype(jnp.float32)
    counts = jnp.sum(onehot, axis=0, keepdims=True)
    ei = jax.lax.broadcasted_iota(jnp.int32, (E, E), 0)
    ej = jax.lax.broadcasted_iota(jnp.int32, (E, E), 1)
    upper = (ei < ej).astype(jnp.float32)
    offs = jax.lax.dot_general(
        counts, upper, (((1,), (0,)), ((), ())),
        preferred_element_type=jnp.float32)
    si = jax.lax.broadcasted_iota(jnp.int32, (S, S), 0)
    sj = jax.lax.broadcasted_iota(jnp.int32, (S, S), 1)
    lower = (sj <= si).astype(jnp.float32)
    pref = jax.lax.dot_general(
        lower, onehot, (((1,), (0,)), ((), ())),
        preferred_element_type=jnp.float32)
    rank = jnp.sum(pref * onehot, axis=1, keepdims=True) - 1.0
    tok_off = jax.lax.dot_general(
        onehot, offs, (((1,), (1,)), ((), ())),
        preferred_element_type=jnp.float32)
    dest = (tok_off + rank).astype(jnp.int32)

    perm = (dest == jax.lax.broadcasted_iota(jnp.int32, (S, S), 1))
    perm = perm.astype(jnp.bfloat16)
    gx = jax.lax.dot_general(
        perm, x.astype(jnp.bfloat16), (((0,), (0,)), ((), ())),
        preferred_element_type=jnp.float32)
    gx_ref[...] = gx.astype(jnp.bfloat16)
    dest_ref[...] = dest
    c128 = jnp.concatenate(
        [counts.astype(jnp.int32), jnp.zeros((1, 128 - E), jnp.int32)], axis=1)
    counts_ref[...] = jnp.concatenate(
        [c128, jnp.zeros((7, 128), jnp.int32)], axis=0)


def _gmm_kernel(tb_ref, te_ref, off_ref,
                gx_ref, wfc_ref, bfc_ref, wproj_ref, bproj_ref, gy_ref):
    i = pl.program_id(0)
    e = te_ref[i]
    b = tb_ref[i]
    start = off_ref[e]
    end = off_ref[e + 1]
    x = gx_ref[...]
    wfc = wfc_ref[0].astype(jnp.bfloat16)
    h = jax.lax.dot_general(
        x, wfc, (((1,), (1,)), ((), ())),
        preferred_element_type=jnp.float32)
    h = jax.nn.gelu(h + bfc_ref[0]).astype(jnp.bfloat16)
    wproj = wproj_ref[0].astype(jnp.bfloat16)
    y = jax.lax.dot_general(
        h, wproj, (((1,), (1,)), ((), ())),
        preferred_element_type=jnp.float32)
    y = y + bproj_ref[0]
    row = b * T + jax.lax.broadcasted_iota(jnp.int32, (T, 1), 0)
    mask = (row >= start) & (row < end)
    gy_ref[...] = jnp.where(mask, y, gy_ref[...])


def _unperm_kernel(dest_ref, gy_ref, out_ref):
    dest = dest_ref[...]
    perm = (dest == jax.lax.broadcasted_iota(jnp.int32, (S, S), 1))
    out_ref[...] = jax.lax.dot_general(
        perm.astype(jnp.float32), gy_ref[...], (((1,), (0,)), ((), ())),
        preferred_element_type=jnp.float32)


@jax.jit
def kernel(hidden_states, router_w, w_fc, b_fc, w_proj, b_proj):
    x = hidden_states.reshape(S, D_MODEL)

    gx, dest, counts_pad = pl.pallas_call(
        _route_kernel,
        out_shape=[
            jax.ShapeDtypeStruct((S, D_MODEL), jnp.bfloat16),
            jax.ShapeDtypeStruct((S, 1), jnp.int32),
            jax.ShapeDtypeStruct((8, 128), jnp.int32),
        ],
    )(x, router_w)

    counts = counts_pad[0, :E]
    offsets = jnp.concatenate(
        [jnp.zeros((1,), jnp.int32), jnp.cumsum(counts, dtype=jnp.int32)])
    b_idx = jnp.arange(NB, dtype=jnp.int32)[:, None]
    start_e = offsets[:-1][None, :]
    end_e = offsets[1:][None, :]
    valid = (end_e > b_idx * T) & (start_e < (b_idx + 1) * T)
    lin = jnp.arange(NB * E, dtype=jnp.int32)
    key = jnp.where(valid.reshape(-1), lin, NB * E)
    order = jnp.sort(key)[:NT]
    is_pad = order >= NB * E
    e_last = jnp.sum((offsets[:E] < S).astype(jnp.int32)) - 1
    tile_b = jnp.where(is_pad, NB - 1, order // E).astype(jnp.int32)
    tile_e = jnp.where(is_pad, e_last, order % E).astype(jnp.int32)

    gy = pl.pallas_call(
        _gmm_kernel,
        grid_spec=pltpu.PrefetchScalarGridSpec(
            num_scalar_prefetch=3,
            grid=(NT,),
            in_specs=[
                pl.BlockSpec((T, D_MODEL), lambda i, tb, te, off: (tb[i], 0)),
                pl.BlockSpec((1, D_FF, D_MODEL),
                             lambda i, tb, te, off: (te[i], 0, 0)),
                pl.BlockSpec((1, 1, D_FF),
                             lambda i, tb, te, off: (te[i], 0, 0)),
                pl.BlockSpec((1, D_MODEL, D_FF),
                             lambda i, tb, te, off: (te[i], 0, 0)),
                pl.BlockSpec((1, 1, D_MODEL),
                             lambda i, tb, te, off: (te[i], 0, 0)),
            ],
            out_specs=pl.BlockSpec((T, D_MODEL), lambda i, tb, te, off: (tb[i], 0)),
        ),
        out_shape=jax.ShapeDtypeStruct((S, D_MODEL), jnp.float32),
    )(tile_b, tile_e, offsets,
      gx,
      w_fc,
      b_fc.reshape(E, 1, D_FF),
      w_proj,
      b_proj.reshape(E, 1, D_MODEL))

    out = pl.pallas_call(
        _unperm_kernel,
        out_shape=jax.ShapeDtypeStruct((S, D_MODEL), jnp.float32),
    )(dest, gy)
    return out.reshape(hidden_states.shape)


# speedup vs baseline: 1.0362x; 1.0362x over previous
import functools

import jax
import jax.numpy as jnp
from jax.experimental import pallas as pl
from jax.experimental.pallas import tpu as pltpu
from jax.experimental.pallas import tpu_sc as plsc

S = 2048
E = 64
D_MODEL = 768
D_FF = 3072
T = 256
QT = 64
NB = S // T
NT = NB + E - 1


def _route_kernel(x_ref, rw_ref, dest_ref, counts_ref):
    x = x_ref[...]
    rw = rw_ref[...]
    logits = jax.lax.dot_general(
        x, rw, (((1,), (1,)), ((), ())), preferred_element_type=jnp.float32)
    m = jnp.max(logits, axis=1, keepdims=True)
    e_iota = jax.lax.broadcasted_iota(jnp.int32, (S, E), 1)
    expert = jnp.min(jnp.where(logits == m, e_iota, E), axis=1, keepdims=True)

    onehot = (e_iota == expert).astype(jnp.float32)
    counts = jnp.sum(onehot, axis=0, keepdims=True)
    ei = jax.lax.broadcasted_iota(jnp.int32, (E, E), 0)
    ej = jax.lax.broadcasted_iota(jnp.int32, (E, E), 1)
    upper = (ei < ej).astype(jnp.float32)
    c_hi = jnp.floor(counts * (1.0 / 128.0))
    c_lo = counts - 128.0 * c_hi
    dots = lambda a, b, dims: jax.lax.dot_general(
        a, b, (dims, ((), ())), preferred_element_type=jnp.float32)
    offs = (128.0 * dots(c_hi, upper, ((1,), (0,)))
            + dots(c_lo, upper, ((1,), (0,))))
    si = jax.lax.broadcasted_iota(jnp.int32, (S, S), 0)
    sj = jax.lax.broadcasted_iota(jnp.int32, (S, S), 1)
    lower = (sj <= si).astype(jnp.float32)
    pref = dots(lower, onehot, ((1,), (0,)))
    rank = jnp.sum(pref * onehot, axis=1, keepdims=True) - 1.0
    o_hi = jnp.floor(offs * (1.0 / 128.0))
    o_lo = offs - 128.0 * o_hi
    tok_off = (128.0 * dots(onehot, o_hi, ((1,), (1,)))
               + dots(onehot, o_lo, ((1,), (1,))))
    dest_f = tok_off + rank
    dest_ref[...] = dest_f.astype(jnp.int32)

    c128 = jnp.concatenate(
        [counts.astype(jnp.int32), jnp.zeros((1, 128 - E), jnp.int32)], axis=1)
    counts_ref[...] = jnp.concatenate(
        [c128, jnp.zeros((7, 128), jnp.int32)], axis=0)


def _sc_gather_body(nc, b_per_w, table_hbm, idx_hbm, out_hbm, idx_v, rows_v, sem):
    wid = jax.lax.axis_index("s") * nc + jax.lax.axis_index("c")
    base = wid * b_per_w
    pltpu.sync_copy(idx_hbm.at[pl.ds(base, b_per_w)], idx_v)
    pltpu.async_copy(table_hbm.at[idx_v], rows_v, sem).wait()
    pltpu.sync_copy(rows_v, out_hbm.at[pl.ds(base, b_per_w)])


def _sc_scatter_body(nc, b_per_w, table_hbm, idx_hbm, out_hbm, idx_v, rows_v, sem):
    wid = jax.lax.axis_index("s") * nc + jax.lax.axis_index("c")
    base = wid * b_per_w
    pltpu.sync_copy(idx_hbm.at[pl.ds(base, b_per_w)], idx_v)
    pltpu.sync_copy(table_hbm.at[pl.ds(base, b_per_w)], rows_v)
    pltpu.async_copy(rows_v, out_hbm.at[idx_v], sem).wait()


def _sc_permute(table, idx, body):
    info = plsc.get_sparse_core_info()
    nw = info.num_cores * info.num_subcores
    n, d = table.shape
    b_per_w = n // nw
    return pl.kernel(
        functools.partial(body, info.num_cores, b_per_w),
        out_type=jax.ShapeDtypeStruct((n, d), table.dtype),
        mesh=plsc.VectorSubcoreMesh(core_axis_name="c", subcore_axis_name="s"),
        scratch_types=[
            pltpu.VMEM((b_per_w,), jnp.int32),
            pltpu.VMEM((b_per_w, d), table.dtype),
            pltpu.SemaphoreType.DMA,
        ],
    )(table, idx)


def _gmm_kernel(tb_ref, te_ref, off_ref,
                gx_ref, wfc_ref, bfc_ref, wproj_ref, bproj_ref, gy_ref):
    i = pl.program_id(0)
    e = te_ref[i]
    b = tb_ref[i]
    start = off_ref[e]
    end = off_ref[e + 1]
    x = gx_ref[...].astype(jnp.bfloat16)
    wfc = wfc_ref[0].astype(jnp.bfloat16)
    h = jax.lax.dot_general(
        x, wfc, (((1,), (1,)), ((), ())),
        preferred_element_type=jnp.float32)
    h = jax.nn.gelu(h + bfc_ref[0]).astype(jnp.bfloat16)
    wproj = wproj_ref[0].astype(jnp.bfloat16)
    y = jax.lax.dot_general(
        h, wproj, (((1,), (1,)), ((), ())),
        preferred_element_type=jnp.float32)
    y = y + bproj_ref[0]
    row = b * T + jax.lax.broadcasted_iota(jnp.int32, (T, 1), 0)
    mask = (row >= start) & (row < end)
    gy_ref[...] = jnp.where(mask, y, gy_ref[...])


@jax.jit
def kernel(hidden_states, router_w, w_fc, b_fc, w_proj, b_proj):
    x = hidden_states.reshape(S, D_MODEL)

    dest, counts_pad = pl.pallas_call(
        _route_kernel,
        out_shape=[
            jax.ShapeDtypeStruct((S, 1), jnp.int32),
            jax.ShapeDtypeStruct((8, 128), jnp.int32),
        ],
    )(x, router_w)

    dest_flat = dest.reshape(S)
    gx = _sc_permute(x, dest_flat, _sc_scatter_body)

    counts = counts_pad[0, :E]
    offsets = jnp.concatenate(
        [jnp.zeros((1,), jnp.int32), jnp.cumsum(counts, dtype=jnp.int32)])
    b_idx = jnp.arange(NB, dtype=jnp.int32)[:, None]
    start_e = offsets[:-1][None, :]
    end_e = offsets[1:][None, :]
    valid = ((end_e > b_idx * T) & (start_e < (b_idx + 1) * T)).reshape(-1)
    p = jnp.cumsum(valid.astype(jnp.int32)) - 1
    ind = valid[:, None] & (p[:, None] == jnp.arange(NT, dtype=jnp.int32)[None, :])
    lin = jnp.arange(NB * E, dtype=jnp.int32)
    tile_lin = jnp.sum(jnp.where(ind, lin[:, None], 0), axis=0)
    nvalid = p[-1] + 1
    is_pad = jnp.arange(NT, dtype=jnp.int32) >= nvalid
    e_last = jnp.sum((offsets[:E] < S).astype(jnp.int32)) - 1
    tile_b = jnp.where(is_pad, NB - 1, tile_lin // E).astype(jnp.int32)
    tile_e = jnp.where(is_pad, e_last, tile_lin % E).astype(jnp.int32)

    gy = pl.pallas_call(
        _gmm_kernel,
        grid_spec=pltpu.PrefetchScalarGridSpec(
            num_scalar_prefetch=3,
            grid=(NT,),
            in_specs=[
                pl.BlockSpec((T, D_MODEL), lambda i, tb, te, off: (tb[i], 0)),
                pl.BlockSpec((1, D_FF, D_MODEL),
                             lambda i, tb, te, off: (te[i], 0, 0)),
                pl.BlockSpec((1, 1, D_FF),
                             lambda i, tb, te, off: (te[i], 0, 0)),
                pl.BlockSpec((1, D_MODEL, D_FF),
                             lambda i, tb, te, off: (te[i], 0, 0)),
                pl.BlockSpec((1, 1, D_MODEL),
                             lambda i, tb, te, off: (te[i], 0, 0)),
            ],
            out_specs=pl.BlockSpec((T, D_MODEL), lambda i, tb, te, off: (tb[i], 0)),
        ),
        out_shape=jax.ShapeDtypeStruct((S, D_MODEL), jnp.float32),
    )(tile_b, tile_e, offsets,
      gx,
      w_fc,
      b_fc.reshape(E, 1, D_FF),
      w_proj,
      b_proj.reshape(E, 1, D_MODEL))

    out = _sc_permute(gy, dest_flat, _sc_gather_body)
    return out.reshape(hidden_states.shape)


# speedup vs baseline: 1.0386x; 1.0023x over previous
import functools

import jax
import jax.numpy as jnp
from jax.experimental import pallas as pl
from jax.experimental.pallas import tpu as pltpu
from jax.experimental.pallas import tpu_sc as plsc

S = 2048
E = 64
D_MODEL = 768
D_FF = 3072
T = 256
NB = S // T
NT = NB + E - 1


def _route_kernel(x_ref, rw_ref, dest_ref, counts_ref):
    x = x_ref[...]
    rw = rw_ref[...]
    logits = jax.lax.dot_general(
        x, rw, (((1,), (1,)), ((), ())), preferred_element_type=jnp.float32)
    m = jnp.max(logits, axis=1, keepdims=True)
    e_iota = jax.lax.broadcasted_iota(jnp.int32, (S, E), 1)
    expert = jnp.min(jnp.where(logits == m, e_iota, E), axis=1, keepdims=True)

    onehot = (e_iota == expert).astype(jnp.float32)
    counts = jnp.sum(onehot, axis=0, keepdims=True)
    ei = jax.lax.broadcasted_iota(jnp.int32, (E, E), 0)
    ej = jax.lax.broadcasted_iota(jnp.int32, (E, E), 1)
    upper = (ei < ej).astype(jnp.float32)
    c_hi = jnp.floor(counts * (1.0 / 128.0))
    c_lo = counts - 128.0 * c_hi
    dots = lambda a, b, dims: jax.lax.dot_general(
        a, b, (dims, ((), ())), preferred_element_type=jnp.float32)
    offs = (128.0 * dots(c_hi, upper, ((1,), (0,)))
            + dots(c_lo, upper, ((1,), (0,))))
    si = jax.lax.broadcasted_iota(jnp.int32, (S, S), 0)
    sj = jax.lax.broadcasted_iota(jnp.int32, (S, S), 1)
    lower = (sj <= si).astype(jnp.float32)
    pref = dots(lower, onehot, ((1,), (0,)))
    rank = jnp.sum(pref * onehot, axis=1, keepdims=True) - 1.0
    o_hi = jnp.floor(offs * (1.0 / 128.0))
    o_lo = offs - 128.0 * o_hi
    tok_off = (128.0 * dots(onehot, o_hi, ((1,), (1,)))
               + dots(onehot, o_lo, ((1,), (1,))))
    dest_f = tok_off + rank
    dest_ref[...] = dest_f.astype(jnp.int32)

    c128 = jnp.concatenate(
        [counts.astype(jnp.int32), jnp.zeros((1, 128 - E), jnp.int32)], axis=1)
    counts_ref[...] = jnp.concatenate(
        [c128, jnp.zeros((7, 128), jnp.int32)], axis=0)


def _sc_gather_body(nc, b_per_w, table_hbm, idx_hbm, out_hbm, idx_v, rows_v, sem):
    wid = jax.lax.axis_index("s") * nc + jax.lax.axis_index("c")
    base = wid * b_per_w
    pltpu.sync_copy(idx_hbm.at[pl.ds(base, b_per_w)], idx_v)
    pltpu.async_copy(table_hbm.at[idx_v], rows_v, sem).wait()
    pltpu.sync_copy(rows_v, out_hbm.at[pl.ds(base, b_per_w)])


def _sc_scatter_body(nc, b_per_w, table_hbm, idx_hbm, out_hbm, idx_v, rows_v, sem):
    wid = jax.lax.axis_index("s") * nc + jax.lax.axis_index("c")
    base = wid * b_per_w
    pltpu.sync_copy(idx_hbm.at[pl.ds(base, b_per_w)], idx_v)
    pltpu.sync_copy(table_hbm.at[pl.ds(base, b_per_w)], rows_v)
    pltpu.async_copy(rows_v, out_hbm.at[idx_v], sem).wait()


def _sc_permute(table, idx, body):
    info = plsc.get_sparse_core_info()
    nw = info.num_cores * info.num_subcores
    n, d = table.shape
    b_per_w = n // nw
    return pl.kernel(
        functools.partial(body, info.num_cores, b_per_w),
        out_type=jax.ShapeDtypeStruct((n, d), table.dtype),
        mesh=plsc.VectorSubcoreMesh(core_axis_name="c", subcore_axis_name="s"),
        scratch_types=[
            pltpu.VMEM((b_per_w,), jnp.int32),
            pltpu.VMEM((b_per_w, d), table.dtype),
            pltpu.SemaphoreType.DMA,
        ],
    )(table, idx)


def _gmm_kernel(tb_ref, te_ref, off_ref,
                gx_ref, wfc_ref, bfc_ref, wproj_ref, bproj_ref, gy_ref):
    i = pl.program_id(0)
    e = te_ref[i]
    b = tb_ref[i]
    start = off_ref[e]
    end = off_ref[e + 1]
    x = gx_ref[...].astype(jnp.bfloat16)
    wfc = wfc_ref[0].astype(jnp.bfloat16)
    h = jax.lax.dot_general(
        x, wfc, (((1,), (1,)), ((), ())),
        preferred_element_type=jnp.float32)
    h = jax.nn.gelu(h + bfc_ref[0]).astype(jnp.bfloat16)
    wproj = wproj_ref[0].astype(jnp.bfloat16)
    y = jax.lax.dot_general(
        h, wproj, (((1,), (1,)), ((), ())),
        preferred_element_type=jnp.float32)
    y = y + bproj_ref[0]
    row = b * T + jax.lax.broadcasted_iota(jnp.int32, (T, 1), 0)
    mask = (row >= start) & (row < end)
    gy_ref[...] = jnp.where(mask, y, gy_ref[...])


@jax.jit
def kernel(hidden_states, router_w, w_fc, b_fc, w_proj, b_proj):
    x = hidden_states.reshape(S, D_MODEL)

    dest, counts_pad = pl.pallas_call(
        _route_kernel,
        out_shape=[
            jax.ShapeDtypeStruct((S, 1), jnp.int32),
            jax.ShapeDtypeStruct((8, 128), jnp.int32),
        ],
    )(x, router_w)

    dest_flat = dest.reshape(S)
    gx = _sc_permute(x, dest_flat, _sc_scatter_body)

    counts = counts_pad[0, :E]
    offsets = jnp.concatenate(
        [jnp.zeros((1,), jnp.int32), jnp.cumsum(counts, dtype=jnp.int32)])
    b_idx = jnp.arange(NB, dtype=jnp.int32)[:, None]
    start_e = offsets[:-1][None, :]
    end_e = offsets[1:][None, :]
    valid = ((end_e > b_idx * T) & (start_e < (b_idx + 1) * T)).reshape(-1)
    p = jnp.cumsum(valid.astype(jnp.int32)) - 1
    ind = valid[:, None] & (p[:, None] == jnp.arange(NT, dtype=jnp.int32)[None, :])
    lin = jnp.arange(NB * E, dtype=jnp.int32)
    tile_lin = jnp.sum(jnp.where(ind, lin[:, None], 0), axis=0)
    nvalid = p[-1] + 1
    is_pad = jnp.arange(NT, dtype=jnp.int32) >= nvalid
    e_last = jnp.sum((offsets[:E] < S).astype(jnp.int32)) - 1
    tile_b = jnp.where(is_pad, NB - 1, tile_lin // E).astype(jnp.int32)
    tile_e = jnp.where(is_pad, e_last, tile_lin % E).astype(jnp.int32)

    gy = pl.pallas_call(
        _gmm_kernel,
        grid_spec=pltpu.PrefetchScalarGridSpec(
            num_scalar_prefetch=3,
            grid=(NT,),
            in_specs=[
                pl.BlockSpec((T, D_MODEL), lambda i, tb, te, off: (tb[i], 0)),
                pl.BlockSpec((1, D_FF, D_MODEL),
                             lambda i, tb, te, off: (te[i], 0, 0)),
                pl.BlockSpec((1, 1, D_FF),
                             lambda i, tb, te, off: (te[i], 0, 0)),
                pl.BlockSpec((1, D_MODEL, D_FF),
                             lambda i, tb, te, off: (te[i], 0, 0)),
                pl.BlockSpec((1, 1, D_MODEL),
                             lambda i, tb, te, off: (te[i], 0, 0)),
            ],
            out_specs=pl.BlockSpec((T, D_MODEL), lambda i, tb, te, off: (tb[i], 0)),
        ),
        out_shape=jax.ShapeDtypeStruct((S, D_MODEL), jnp.float32),
    )(tile_b, tile_e, offsets,
      gx,
      w_fc,
      b_fc.reshape(E, 1, D_FF),
      w_proj,
      b_proj.reshape(E, 1, D_MODEL))

    out = _sc_permute(gy, dest_flat, _sc_gather_body)
    return out.reshape(hidden_states.shape)
